# Initial kernel scaffold; baseline (speedup 1.0000x reference)
import jax
import jax.numpy as jnp
from jax.experimental import pallas as pl


def kernel(x, edge_index, edge_attr, batch, W1l, W1r, b1, g1, be1, W2l, W2r, b2, Wf1, bf1, g2, be2, Wf2, bf2):
    raise NotImplementedError("write your pallas kernel here")


# speedup vs baseline: 7.4413x; 7.4413x over previous
import functools

import jax
import jax.numpy as jnp
from jax import lax
from jax.experimental import pallas as pl
from jax.experimental.pallas import tpu as pltpu
from jax.experimental.pallas import tpu_sc as plsc

N_NODES = 10000
N_PAD = 10240
E_EDGES = 160000
E_PAD = 163840
N_TILES = 32
CHUNKS = 40
CHUNK = 128
ROWS_PER_TILE = N_PAD // 16
NEG_INF = -3.0e38


def _stage1_body(x_ref, w_ref, t1_ref, xr_ref):
    i = pl.program_id(0)
    xw = jnp.dot(x_ref[...], w_ref[...], preferred_element_type=jnp.float32)
    rows = i * 256 + lax.broadcasted_iota(jnp.int32, (256, 1), 0)
    valid = rows < N_NODES
    cols = lax.broadcasted_iota(jnp.int32, (256, 32), 1)
    ones_col = jnp.where(jnp.logical_and(valid, cols == 16), 1.0, 0.0)
    t1_ref[...] = xw[:, :32] + ones_col
    xr_ref[...] = xw[:, 32:48]


def _stage1(x_pad, w_all):
    return pl.pallas_call(
        _stage1_body,
        grid=(N_PAD // 256,),
        in_specs=[
            pl.BlockSpec((256, 256), lambda i: (i, 0)),
            pl.BlockSpec((256, 48), lambda i: (0, 0)),
        ],
        out_specs=[
            pl.BlockSpec((256, 32), lambda i: (i, 0)),
            pl.BlockSpec((256, 16), lambda i: (i, 0)),
        ],
        out_shape=[
            jax.ShapeDtypeStruct((N_PAD, 32), jnp.float32),
            jax.ShapeDtypeStruct((N_PAD, 16), jnp.float32),
        ],
    )(x_pad, w_all)


@functools.lru_cache(maxsize=None)
def _make_sc_agg(width):
    mesh = plsc.VectorSubcoreMesh(core_axis_name="c", subcore_axis_name="s")

    @functools.partial(
        pl.kernel,
        mesh=mesh,
        out_type=jax.ShapeDtypeStruct((2, N_PAD, width), jnp.float32),
        scratch_types=[
            pltpu.VMEM((CHUNKS, CHUNK), jnp.int32),
            pltpu.VMEM((CHUNKS, CHUNK), jnp.int32),
            pltpu.VMEM((CHUNK, width), jnp.float32),
            pltpu.VMEM_SHARED((N_PAD, width), jnp.float32),
            pltpu.SemaphoreType.DMA,
        ],
        compiler_params=pltpu.CompilerParams(use_tc_tiling_on_sc=False),
    )
    def sc_agg(table_hbm, src_hbm, dst_hbm, zero_hbm, out_hbm,
               src_v, dst_v, rows_v, acc_sh, sem):
        c = lax.axis_index("c")
        s = lax.axis_index("s")
        wid = s * 2 + c
        row0 = s * ROWS_PER_TILE
        pltpu.sync_copy(zero_hbm.at[pl.ds(row0, ROWS_PER_TILE)],
                        acc_sh.at[pl.ds(row0, ROWS_PER_TILE)])
        pltpu.sync_copy(src_hbm.at[wid], src_v)
        pltpu.sync_copy(dst_hbm.at[wid], dst_v)
        plsc.subcore_barrier()

        def body(j, carry):
            pltpu.async_copy(table_hbm.at[src_v.at[j]], rows_v, sem).wait()
            pltpu.sync_copy(rows_v, acc_sh.at[dst_v.at[j]], add=True)
            return carry

        lax.fori_loop(0, CHUNKS, body, 0)
        plsc.subcore_barrier()
        pltpu.sync_copy(acc_sh.at[pl.ds(row0, ROWS_PER_TILE)],
                        out_hbm.at[c, pl.ds(row0, ROWS_PER_TILE)])

    return sc_agg


def _sc_agg(table, src3, dst3, zero, width):
    return _make_sc_agg(width)(table, src3, dst3, zero)


def _stage2_body(a0_ref, a1_ref, xr_ref, w2_ref, b1_ref, g1_ref, be1_ref,
                 b2_ref, t2_ref, r2_ref, inv_ref):
    i = pl.program_id(0)
    acc = a0_ref[...] + a1_ref[...]
    agg = acc[:, :16]
    deg = acc[:, 16:17]
    invd = 1.0 / jnp.maximum(deg, 1.0)
    pre = agg * invd + b1_ref[...] + xr_ref[...]
    h = jnp.maximum(pre, 0.0)
    mu = jnp.mean(h, axis=1, keepdims=True)
    var = jnp.mean((h - mu) ** 2, axis=1, keepdims=True)
    hn = (h - mu) * lax.rsqrt(var + 1e-5) * g1_ref[...] + be1_ref[...]
    hw = jnp.dot(hn, w2_ref[...], preferred_element_type=jnp.float32)
    rows = i * 256 + lax.broadcasted_iota(jnp.int32, (256, 1), 0)
    valid = rows < N_NODES
    t2_ref[...] = jnp.where(valid, hw[:, :16], 0.0)
    r2_ref[...] = hw[:, 16:32] + b2_ref[...]
    inv_ref[...] = invd


def _stage2(acc0, acc1, xr, w2_all, b1, g1, be1, b2):
    row = lambda i: (i, 0)
    full = lambda i: (0, 0)
    return pl.pallas_call(
        _stage2_body,
        grid=(N_PAD // 256,),
        in_specs=[
            pl.BlockSpec((256, 32), row),
            pl.BlockSpec((256, 32), row),
            pl.BlockSpec((256, 16), row),
            pl.BlockSpec((16, 32), full),
            pl.BlockSpec((1, 16), full),
            pl.BlockSpec((1, 16), full),
            pl.BlockSpec((1, 16), full),
            pl.BlockSpec((1, 16), full),
        ],
        out_specs=[
            pl.BlockSpec((256, 16), row),
            pl.BlockSpec((256, 16), row),
            pl.BlockSpec((256, 1), row),
        ],
        out_shape=[
            jax.ShapeDtypeStruct((N_PAD, 16), jnp.float32),
            jax.ShapeDtypeStruct((N_PAD, 16), jnp.float32),
            jax.ShapeDtypeStruct((N_PAD, 1), jnp.float32),
        ],
    )(acc0, acc1, xr, w2_all, b1, g1, be1, b2)


def _stage3_body(a0_ref, a1_ref, r2_ref, inv_ref, batch_ref,
                 wf1t_ref, bf1_ref, g2_ref, be2_ref, wf2t_ref, bf2_ref,
                 out_ref, pool_ref):
    i = pl.program_id(0)
    nblk = pl.num_programs(0)
    h2 = jnp.maximum((a0_ref[...] + a1_ref[...]) * inv_ref[...] + r2_ref[...],
                     0.0)
    b = batch_ref[...]

    @pl.when(i == 0)
    def _init():
        pool_ref[...] = jnp.full((16, 64), NEG_INF, jnp.float32)

    cols = []
    for g in range(64):
        cand = jnp.where(b == g, h2, NEG_INF)
        cols.append(jnp.max(cand, axis=1, keepdims=True))
    blk = jnp.concatenate(cols, axis=1)
    pool_ref[...] = jnp.maximum(pool_ref[...], blk)

    @pl.when(i == nblk - 1)
    def _head():
        pooled = pool_ref[...]
        z = jnp.dot(wf1t_ref[...], pooled,
                    preferred_element_type=jnp.float32) + bf1_ref[...]
        mu = jnp.mean(z, axis=0, keepdims=True)
        var = jnp.mean((z - mu) ** 2, axis=0, keepdims=True)
        z = (z - mu) * lax.rsqrt(var + 1e-5) * g2_ref[...] + be2_ref[...]
        z = jnp.maximum(z, 0.0)
        z = jnp.dot(wf2t_ref[...], z,
                    preferred_element_type=jnp.float32) + bf2_ref[...]
        m = jnp.max(z, axis=0, keepdims=True)
        lse = m + jnp.log(jnp.sum(jnp.exp(z - m), axis=0, keepdims=True))
        out_ref[...] = z - lse


def _stage3(a0t, a1t, r2t, invt, batch_t, wf1t, bf1, g2, be2, wf2t, bf2):
    row = lambda i: (0, i)
    full = lambda i: (0, 0)
    out_t, _ = pl.pallas_call(
        _stage3_body,
        grid=(N_PAD // 1024,),
        in_specs=[
            pl.BlockSpec((16, 1024), row),
            pl.BlockSpec((16, 1024), row),
            pl.BlockSpec((16, 1024), row),
            pl.BlockSpec((1, 1024), row),
            pl.BlockSpec((1, 1024), row),
            pl.BlockSpec((32, 16), full),
            pl.BlockSpec((32, 1), full),
            pl.BlockSpec((32, 1), full),
            pl.BlockSpec((32, 1), full),
            pl.BlockSpec((2, 32), full),
            pl.BlockSpec((2, 1), full),
        ],
        out_specs=[
            pl.BlockSpec((2, 64), full),
            pl.BlockSpec((16, 64), full),
        ],
        out_shape=[
            jax.ShapeDtypeStruct((2, 64), jnp.float32),
            jax.ShapeDtypeStruct((16, 64), jnp.float32),
        ],
    )(a0t, a1t, r2t, invt, batch_t, wf1t, bf1, g2, be2, wf2t, bf2)
    return out_t.T


def kernel(x, edge_index, edge_attr, batch, W1l, W1r, b1, g1, be1,
           W2l, W2r, b2, Wf1, bf1, g2, be2, Wf2, bf2):
    x_pad = jnp.zeros((N_PAD, 256), jnp.float32).at[:N_NODES].set(x)
    src = jnp.full((E_PAD,), N_NODES, jnp.int32).at[:E_EDGES].set(
        edge_index[0].astype(jnp.int32))
    dst = jnp.full((E_PAD,), N_NODES, jnp.int32).at[:E_EDGES].set(
        edge_index[1].astype(jnp.int32))
    src3 = src.reshape(N_TILES, CHUNKS, CHUNK)
    dst3 = dst.reshape(N_TILES, CHUNKS, CHUNK)
    batch_t = jnp.full((1, N_PAD), 64, jnp.int32).at[0, :N_NODES].set(
        batch.astype(jnp.int32))
    w_all = jnp.concatenate(
        [W1l, jnp.zeros((256, 16), jnp.float32), W1r], axis=1)
    w2_all = jnp.concatenate([W2l, W2r], axis=1)
    zero32 = jnp.zeros((N_PAD, 32), jnp.float32)
    zero16 = jnp.zeros((N_PAD, 16), jnp.float32)

    table1, xr = _stage1(x_pad, w_all)
    acc1 = _sc_agg(table1, src3, dst3, zero32, 32)
    table2, rest2, invdeg = _stage2(
        acc1[0], acc1[1], xr, w2_all,
        b1.reshape(1, 16), g1.reshape(1, 16), be1.reshape(1, 16),
        b2.reshape(1, 16))
    acc2 = _sc_agg(table2, src3, dst3, zero16, 16)
    return _stage3(acc2[0].T, acc2[1].T, rest2.T, invdeg.reshape(1, N_PAD),
                   batch_t, Wf1.T, bf1.reshape(32, 1), g2.reshape(32, 1),
                   be2.reshape(32, 1), Wf2.T, bf2.reshape(2, 1))


# speedup vs baseline: 9.3387x; 1.2550x over previous
import functools

import jax
import jax.numpy as jnp
from jax import lax
from jax.experimental import pallas as pl
from jax.experimental.pallas import tpu as pltpu
from jax.experimental.pallas import tpu_sc as plsc

N_NODES = 10000
N_PAD = 10240
E_EDGES = 160000
E_PAD = 163840
N_TILES = 32
CHUNKS = 40
CHUNK = 128
ROWS_PER_TILE = N_PAD // 16
NEG_INF = -3.0e38


def _stage1_body(x_ref, w_ref, t1_ref, xr_ref):
    xw = jnp.dot(x_ref[...], w_ref[...], preferred_element_type=jnp.float32)
    cols = lax.broadcasted_iota(jnp.int32, (400, 32), 1)
    t1_ref[...] = xw[:, :32] + jnp.where(cols == 16, 1.0, 0.0)
    xr_ref[...] = xw[:, 32:48]


def _stage1(x, w_all):
    return pl.pallas_call(
        _stage1_body,
        grid=(N_NODES // 400,),
        in_specs=[
            pl.BlockSpec((400, 256), lambda i: (i, 0)),
            pl.BlockSpec((256, 48), lambda i: (0, 0)),
        ],
        out_specs=[
            pl.BlockSpec((400, 32), lambda i: (i, 0)),
            pl.BlockSpec((400, 16), lambda i: (i, 0)),
        ],
        out_shape=[
            jax.ShapeDtypeStruct((N_PAD, 32), jnp.float32),
            jax.ShapeDtypeStruct((N_PAD, 16), jnp.float32),
        ],
    )(x, w_all)


@functools.lru_cache(maxsize=None)
def _make_sc_agg(width):
    mesh = plsc.VectorSubcoreMesh(core_axis_name="c", subcore_axis_name="s")

    @functools.partial(
        pl.kernel,
        mesh=mesh,
        out_type=jax.ShapeDtypeStruct((2, N_PAD, width), jnp.float32),
        scratch_types=[
            pltpu.VMEM((CHUNKS, CHUNK), jnp.int32),
            pltpu.VMEM((CHUNKS, CHUNK), jnp.int32),
            pltpu.VMEM((CHUNK, width), jnp.float32),
            pltpu.VMEM((CHUNK, width), jnp.float32),
            pltpu.VMEM_SHARED((N_PAD, width), jnp.float32),
            pltpu.SemaphoreType.DMA,
            pltpu.SemaphoreType.DMA,
        ],
        compiler_params=pltpu.CompilerParams(use_tc_tiling_on_sc=False),
    )
    def sc_agg(table_hbm, src_hbm, dst_hbm, zero_hbm, out_hbm,
               src_v, dst_v, rows0, rows1, acc_sh, sem0, sem1):
        c = lax.axis_index("c")
        s = lax.axis_index("s")
        wid = s * 2 + c
        row0 = s * ROWS_PER_TILE
        pltpu.sync_copy(zero_hbm.at[pl.ds(row0, ROWS_PER_TILE)],
                        acc_sh.at[pl.ds(row0, ROWS_PER_TILE)])
        pltpu.sync_copy(src_hbm.at[wid], src_v)
        pltpu.sync_copy(dst_hbm.at[wid], dst_v)
        plsc.subcore_barrier()

        bufs = ((rows0, sem0), (rows1, sem1))
        pltpu.async_copy(table_hbm.at[src_v.at[0]], rows0, sem0)
        pltpu.async_copy(table_hbm.at[src_v.at[1]], rows1, sem1)

        def body(i, carry):
            for b, (rv, sm) in enumerate(bufs):
                j = 2 * i + b
                pltpu.make_async_copy(table_hbm.at[src_v.at[0]],
                                      rv, sm).wait()
                pltpu.sync_copy(rv, acc_sh.at[dst_v.at[j]], add=True)

                @pl.when(j + 2 < CHUNKS)
                def _prefetch():
                    pltpu.async_copy(table_hbm.at[src_v.at[j + 2]], rv, sm)
            return carry

        lax.fori_loop(0, CHUNKS // 2, body, 0)
        plsc.subcore_barrier()
        pltpu.sync_copy(acc_sh.at[pl.ds(row0, ROWS_PER_TILE)],
                        out_hbm.at[c, pl.ds(row0, ROWS_PER_TILE)])

    return sc_agg


def _sc_agg(table, src3, dst3, zero, width):
    return _make_sc_agg(width)(table, src3, dst3, zero)


def _stage2_body(a0_ref, a1_ref, xr_ref, w2_ref, b1_ref, g1_ref, be1_ref,
                 b2_ref, t2_ref, r2_ref, inv_ref):
    i = pl.program_id(0)
    acc = a0_ref[...] + a1_ref[...]
    agg = acc[:, :16]
    deg = acc[:, 16:17]
    invd = 1.0 / jnp.maximum(deg, 1.0)
    pre = agg * invd + b1_ref[...] + xr_ref[...]
    h = jnp.maximum(pre, 0.0)
    mu = jnp.mean(h, axis=1, keepdims=True)
    var = jnp.mean((h - mu) ** 2, axis=1, keepdims=True)
    hn = (h - mu) * lax.rsqrt(var + 1e-5) * g1_ref[...] + be1_ref[...]
    hw = jnp.dot(hn, w2_ref[...], preferred_element_type=jnp.float32)
    rows = i * 256 + lax.broadcasted_iota(jnp.int32, (256, 1), 0)
    valid = rows < N_NODES
    t2_ref[...] = jnp.where(valid, hw[:, :16], 0.0)
    r2_ref[...] = hw[:, 16:32] + b2_ref[...]
    inv_ref[...] = invd


def _stage2(acc0, acc1, xr, w2_all, b1, g1, be1, b2):
    row = lambda i: (i, 0)
    full = lambda i: (0, 0)
    return pl.pallas_call(
        _stage2_body,
        grid=(N_PAD // 256,),
        in_specs=[
            pl.BlockSpec((256, 32), row),
            pl.BlockSpec((256, 32), row),
            pl.BlockSpec((256, 16), row),
            pl.BlockSpec((16, 32), full),
            pl.BlockSpec((1, 16), full),
            pl.BlockSpec((1, 16), full),
            pl.BlockSpec((1, 16), full),
            pl.BlockSpec((1, 16), full),
        ],
        out_specs=[
            pl.BlockSpec((256, 16), row),
            pl.BlockSpec((256, 16), row),
            pl.BlockSpec((256, 1), row),
        ],
        out_shape=[
            jax.ShapeDtypeStruct((N_PAD, 16), jnp.float32),
            jax.ShapeDtypeStruct((N_PAD, 16), jnp.float32),
            jax.ShapeDtypeStruct((N_PAD, 1), jnp.float32),
        ],
    )(acc0, acc1, xr, w2_all, b1, g1, be1, b2)


def _stage3_body(a0_ref, a1_ref, r2_ref, inv_ref, batch_ref,
                 wf1t_ref, bf1_ref, g2_ref, be2_ref, wf2t_ref, bf2_ref,
                 out_ref, pool_ref):
    i = pl.program_id(0)
    nblk = pl.num_programs(0)
    h2n = jnp.maximum(
        (a0_ref[...] + a1_ref[...]) * inv_ref[...] + r2_ref[...],
        0.0)
    h2 = h2n.T
    b = batch_ref[...]

    @pl.when(i == 0)
    def _init():
        pool_ref[...] = jnp.full((16, 64), NEG_INF, jnp.float32)

    cols = []
    for g in range(64):
        cand = jnp.where(b == g, h2, NEG_INF)
        cols.append(jnp.max(cand, axis=1, keepdims=True))
    blk = jnp.concatenate(cols, axis=1)
    pool_ref[...] = jnp.maximum(pool_ref[...], blk)

    @pl.when(i == nblk - 1)
    def _head():
        pooled = pool_ref[...]
        z = jnp.dot(wf1t_ref[...], pooled,
                    preferred_element_type=jnp.float32) + bf1_ref[...]
        mu = jnp.mean(z, axis=0, keepdims=True)
        var = jnp.mean((z - mu) ** 2, axis=0, keepdims=True)
        z = (z - mu) * lax.rsqrt(var + 1e-5) * g2_ref[...] + be2_ref[...]
        z = jnp.maximum(z, 0.0)
        z = jnp.dot(wf2t_ref[...], z,
                    preferred_element_type=jnp.float32) + bf2_ref[...]
        m = jnp.max(z, axis=0, keepdims=True)
        lse = m + jnp.log(jnp.sum(jnp.exp(z - m), axis=0, keepdims=True))
        out_ref[...] = z - lse


def _stage3(a0, a1, r2, inv, batch_t, wf1t, bf1, g2, be2, wf2t, bf2):
    row = lambda i: (i, 0)
    rowt = lambda i: (0, i)
    full = lambda i: (0, 0)
    out_t, _ = pl.pallas_call(
        _stage3_body,
        grid=(N_PAD // 1024,),
        in_specs=[
            pl.BlockSpec((1024, 16), row),
            pl.BlockSpec((1024, 16), row),
            pl.BlockSpec((1024, 16), row),
            pl.BlockSpec((1024, 1), row),
            pl.BlockSpec((1, 1024), rowt),
            pl.BlockSpec((32, 16), full),
            pl.BlockSpec((32, 1), full),
            pl.BlockSpec((32, 1), full),
            pl.BlockSpec((32, 1), full),
            pl.BlockSpec((2, 32), full),
            pl.BlockSpec((2, 1), full),
        ],
        out_specs=[
            pl.BlockSpec((2, 64), full),
            pl.BlockSpec((16, 64), full),
        ],
        out_shape=[
            jax.ShapeDtypeStruct((2, 64), jnp.float32),
            jax.ShapeDtypeStruct((16, 64), jnp.float32),
        ],
    )(a0, a1, r2, inv, batch_t, wf1t, bf1, g2, be2, wf2t, bf2)
    return out_t.T


def kernel(x, edge_index, edge_attr, batch, W1l, W1r, b1, g1, be1,
           W2l, W2r, b2, Wf1, bf1, g2, be2, Wf2, bf2):
    src = jnp.full((E_PAD,), N_NODES, jnp.int32).at[:E_EDGES].set(
        edge_index[0].astype(jnp.int32))
    dst = jnp.full((E_PAD,), N_NODES, jnp.int32).at[:E_EDGES].set(
        edge_index[1].astype(jnp.int32))
    src3 = src.reshape(N_TILES, CHUNKS, CHUNK)
    dst3 = dst.reshape(N_TILES, CHUNKS, CHUNK)
    batch_t = jnp.full((1, N_PAD), 64, jnp.int32).at[0, :N_NODES].set(
        batch.astype(jnp.int32))
    w_all = jnp.concatenate(
        [W1l, jnp.zeros((256, 16), jnp.float32), W1r], axis=1)
    w2_all = jnp.concatenate([W2l, W2r], axis=1)
    zero32 = jnp.zeros((N_PAD, 32), jnp.float32)
    zero16 = jnp.zeros((N_PAD, 16), jnp.float32)

    table1, xr = _stage1(x, w_all)
    acc1 = _sc_agg(table1, src3, dst3, zero32, 32)
    table2, rest2, invdeg = _stage2(
        acc1[0], acc1[1], xr, w2_all,
        b1.reshape(1, 16), g1.reshape(1, 16), be1.reshape(1, 16),
        b2.reshape(1, 16))
    acc2 = _sc_agg(table2, src3, dst3, zero16, 16)
    return _stage3(acc2[0], acc2[1], rest2, invdeg, batch_t,
                   Wf1.T, bf1.reshape(32, 1), g2.reshape(32, 1),
                   be2.reshape(32, 1), Wf2.T, bf2.reshape(2, 1))
